# baseline (speedup 1.0000x reference)
import jax
import jax.numpy as jnp
from jax.experimental import pallas as pl
from jax.experimental.pallas import tpu as pltpu

_BN_EPS = 1e-5


def _make_fused_kernel(B, G, H, W, N):
    M = H * W

    def _fused(x_ref, w_ref, gb_ref, out_ref, conv_v, stats_v, ss_v):
        C = x_ref.shape[2]
        i = pl.program_id(0)
        phase0 = i < G

        m = jax.lax.broadcasted_iota(jnp.int32, (M, 1), 0)
        col = jax.lax.rem(m, W)
        masks = {
            -1: col > 0,
            0: None,
            1: col < W - 1,
        }
        rowm = {
            -1: m >= W,
            0: None,
            1: m < M - W,
        }

        @pl.when(phase0)
        def _():
            @pl.when(i == 0)
            def _():
                stats_v[...] = jnp.zeros((8, C), jnp.float32)

            st = stats_v[...]
            for b in range(B):
                x = x_ref[b].astype(jnp.bfloat16)
                parts = []
                for ky in range(3):
                    dy = ky - 1
                    for kx in range(3):
                        dx = kx - 1
                        s = dy * W + dx
                        t = x if s == 0 else jnp.roll(x, -s, axis=0)
                        mask = None
                        if rowm[dy] is not None and masks[dx] is not None:
                            mask = jnp.logical_and(rowm[dy], masks[dx])
                        elif rowm[dy] is not None:
                            mask = rowm[dy]
                        elif masks[dx] is not None:
                            mask = masks[dx]
                        if mask is not None:
                            t = jnp.where(mask, t, jnp.bfloat16(0))
                        parts.append(t)
                patch = jnp.concatenate(parts, axis=1)
                mxu = b % 2
                kt_n = w_ref.shape[0] // 256
                for kt in range(kt_n):
                    lhs = patch[:, kt * 256:(kt + 1) * 256]
                    if (kt + 1) * 256 > 9 * C:
                        lhs = jnp.concatenate(
                            [patch[:, kt * 256:9 * C],
                             jnp.zeros((M, (kt + 1) * 256 - 9 * C),
                                       jnp.bfloat16)], axis=1)
                    pltpu.matmul_push_rhs(
                        w_ref[kt * 256:(kt + 1) * 256, :],
                        staging_register=kt % 2, mxu_index=mxu)
                    pltpu.matmul_acc_lhs(
                        acc_addr=0, lhs=lhs, mxu_index=mxu,
                        load_staged_rhs=kt % 2)
                acc = pltpu.matmul_pop(
                    acc_addr=0, shape=(M, 256), dtype=jnp.float32,
                    mxu_index=mxu)[:, :C]
                conv_v[i * B + b] = acc.astype(jnp.bfloat16)
                s1 = jnp.sum(acc, axis=0, keepdims=True)
                s2 = jnp.sum(acc * acc, axis=0, keepdims=True)
                st = st + jnp.concatenate(
                    [s1, s2, jnp.zeros((6, C), jnp.float32)], axis=0)
            stats_v[...] = st

        @pl.when(jnp.logical_not(phase0))
        def _():
            @pl.when(i == G)
            def _():
                st = stats_v[...]
                mean = st[0:1, :] / (N * M)
                var = jnp.maximum(st[1:2, :] / (N * M) - mean * mean, 0.0)
                scale = gb_ref[0:1, :] * jax.lax.rsqrt(var + _BN_EPS)
                shift = gb_ref[1:2, :] - mean * scale
                ss_v[...] = jnp.concatenate(
                    [scale, shift, jnp.zeros((6, C), jnp.float32)], axis=0)

            g = i - G
            scale = ss_v[0:1, :]
            shift = ss_v[1:2, :]
            for b in range(B):
                out_ref[b] = (conv_v[g * B + b].astype(jnp.float32) * scale
                              + shift + x_ref[b])

    return _fused


@jax.jit
def _residual_block_opt(x_nchw, w_oihw, gamma, beta):
    N, C, H, W = x_nchw.shape
    M = H * W
    xm = jnp.transpose(x_nchw, (0, 2, 3, 1)).reshape(N, M, C)

    K9 = 9 * C
    Kp = ((K9 + 255) // 256) * 256
    wk = jnp.transpose(w_oihw, (2, 3, 1, 0)).reshape(K9, C).astype(jnp.bfloat16)
    wk = jnp.pad(wk, ((0, Kp - K9), (0, 256 - C)))
    gb = jnp.concatenate(
        [gamma.reshape(1, C).astype(jnp.float32),
         beta.reshape(1, C).astype(jnp.float32),
         jnp.zeros((6, C), jnp.float32)], axis=0)

    B = 8 if N % 8 == 0 else 1
    G = N // B

    out = pl.pallas_call(
        _make_fused_kernel(B, G, H, W, N),
        grid=(2 * G,),
        in_specs=[
            pl.BlockSpec((B, M, C), lambda i: (jnp.where(i < G, i, i - G), 0, 0)),
            pl.BlockSpec((Kp, 256), lambda i: (0, 0)),
            pl.BlockSpec((8, C), lambda i: (0, 0)),
        ],
        out_specs=pl.BlockSpec(
            (B, M, C), lambda i: (jnp.where(i < G, 0, i - G), 0, 0)),
        out_shape=jax.ShapeDtypeStruct((N, M, C), x_nchw.dtype),
        scratch_shapes=[
            pltpu.VMEM((N, M, C), jnp.bfloat16),
            pltpu.VMEM((8, C), jnp.float32),
            pltpu.VMEM((8, C), jnp.float32),
        ],
        compiler_params=pltpu.CompilerParams(
            dimension_semantics=("arbitrary",),
            vmem_limit_bytes=56 * 1024 * 1024),
    )(xm, wk, gb)

    return jnp.transpose(out.reshape(N, H, W, C), (0, 3, 1, 2))


def kernel(x_nchw, w_oihw, bias, gamma, beta):
    del bias
    return _residual_block_opt(x_nchw, w_oihw, gamma, beta)


# speedup vs baseline: 1.2321x; 1.2321x over previous
import jax
import jax.numpy as jnp
from jax.experimental import pallas as pl
from jax.experimental.pallas import tpu as pltpu

_BN_EPS = 1e-5


def _make_fused_kernel(B, G, H, W, N):
    M = H * W

    def _fused(x_ref, w_ref, gb_ref, out_ref, conv_v, xb_v, stats_v, ss_v):
        C = x_ref.shape[2]
        i = pl.program_id(0)
        phase0 = i < G

        m = jax.lax.broadcasted_iota(jnp.int32, (M, 1), 0)
        col = jax.lax.rem(m, W)
        masks = {
            -1: col > 0,
            0: None,
            1: col < W - 1,
        }
        rowm = {
            -1: m >= W,
            0: None,
            1: m < M - W,
        }

        @pl.when(phase0)
        def _():
            @pl.when(i == 0)
            def _():
                stats_v[...] = jnp.zeros((8, C), jnp.float32)

            st = stats_v[...]
            for b in range(B):
                x = x_ref[b].astype(jnp.bfloat16)
                xb_v[i * B + b] = x
                parts = []
                for ky in range(3):
                    dy = ky - 1
                    for kx in range(3):
                        dx = kx - 1
                        s = dy * W + dx
                        t = x if s == 0 else jnp.roll(x, -s, axis=0)
                        mask = None
                        if rowm[dy] is not None and masks[dx] is not None:
                            mask = jnp.logical_and(rowm[dy], masks[dx])
                        elif rowm[dy] is not None:
                            mask = rowm[dy]
                        elif masks[dx] is not None:
                            mask = masks[dx]
                        if mask is not None:
                            t = jnp.where(mask, t, jnp.bfloat16(0))
                        parts.append(t)
                patch = jnp.concatenate(parts, axis=1)
                acc = jnp.dot(patch, w_ref[...],
                              preferred_element_type=jnp.float32)
                conv_v[i * B + b] = acc.astype(jnp.bfloat16)
                s1 = jnp.sum(acc, axis=0, keepdims=True)
                s2 = jnp.sum(acc * acc, axis=0, keepdims=True)
                st = st + jnp.concatenate(
                    [s1, s2, jnp.zeros((6, C), jnp.float32)], axis=0)
            stats_v[...] = st

        @pl.when(jnp.logical_not(phase0))
        def _():
            @pl.when(i == G)
            def _():
                st = stats_v[...]
                mean = st[0:1, :] / (N * M)
                var = jnp.maximum(st[1:2, :] / (N * M) - mean * mean, 0.0)
                scale = gb_ref[0:1, :] * jax.lax.rsqrt(var + _BN_EPS)
                shift = gb_ref[1:2, :] - mean * scale
                ss_v[...] = jnp.concatenate(
                    [scale, shift, jnp.zeros((6, C), jnp.float32)], axis=0)

            g = i - G
            scale = ss_v[0:1, :]
            shift = ss_v[1:2, :]
            for b in range(B):
                out_ref[b] = (conv_v[g * B + b].astype(jnp.float32) * scale
                              + shift + xb_v[g * B + b].astype(jnp.float32))

    return _fused


@jax.jit
def _residual_block_opt(x_nchw, w_oihw, gamma, beta):
    N, C, H, W = x_nchw.shape
    M = H * W
    xm = jnp.transpose(x_nchw, (0, 2, 3, 1)).reshape(N, M, C)

    wk = jnp.transpose(w_oihw, (2, 3, 1, 0)).reshape(9 * C, C).astype(jnp.bfloat16)
    gb = jnp.concatenate(
        [gamma.reshape(1, C).astype(jnp.float32),
         beta.reshape(1, C).astype(jnp.float32),
         jnp.zeros((6, C), jnp.float32)], axis=0)

    B = 8 if N % 8 == 0 else 1
    G = N // B

    out = pl.pallas_call(
        _make_fused_kernel(B, G, H, W, N),
        grid=(2 * G,),
        in_specs=[
            pl.BlockSpec((B, M, C), lambda i: (jnp.where(i < G, i, G - 1), 0, 0)),
            pl.BlockSpec((9 * C, C), lambda i: (0, 0)),
            pl.BlockSpec((8, C), lambda i: (0, 0)),
        ],
        out_specs=pl.BlockSpec(
            (B, M, C), lambda i: (jnp.where(i < G, 0, i - G), 0, 0)),
        out_shape=jax.ShapeDtypeStruct((N, M, C), x_nchw.dtype),
        scratch_shapes=[
            pltpu.VMEM((N, M, C), jnp.bfloat16),
            pltpu.VMEM((N, M, C), jnp.bfloat16),
            pltpu.VMEM((8, C), jnp.float32),
            pltpu.VMEM((8, C), jnp.float32),
        ],
        compiler_params=pltpu.CompilerParams(
            dimension_semantics=("arbitrary",),
            vmem_limit_bytes=56 * 1024 * 1024),
    )(xm, wk, gb)

    return jnp.transpose(out.reshape(N, H, W, C), (0, 3, 1, 2))


def kernel(x_nchw, w_oihw, bias, gamma, beta):
    del bias
    return _residual_block_opt(x_nchw, w_oihw, gamma, beta)
